# Initial kernel scaffold; baseline (speedup 1.0000x reference)
import jax
import jax.numpy as jnp
from jax.experimental import pallas as pl


def kernel(x, edge_index, edge_attr, timestamps, We1, be1, We2, be2, Wn1, bn1, Wn2, bn2, Wt, bt):
    raise NotImplementedError("write your pallas kernel here")


# speedup vs baseline: 2.3625x; 2.3625x over previous
import functools

import jax
import jax.numpy as jnp
from jax import lax
from jax.experimental import pallas as pl
from jax.experimental.pallas import tpu as pltpu
from jax.experimental.pallas import tpu_sc as plsc

NUM_SC = 2
NUM_TEC = 16
N_TILES = NUM_SC * NUM_TEC
CH = 80


def _xa_body(x_ref, w_ref, o_ref):
    o_ref[...] = jnp.dot(x_ref[...], w_ref[...], preferred_element_type=jnp.float32)


def _msg_body(gx_ref, ea_ref, ts_ref, We1_ref, be1_ref, We2_ref, be2_ref,
              Wt_ref, bt_ref, Wn1b_ref, bn1_ref, Wn2_ref, bn2_ref, msg_ref):
    h = jnp.maximum(
        jnp.dot(ea_ref[...], We1_ref[...], preferred_element_type=jnp.float32)
        + be1_ref[...], 0.0)
    ef = jnp.dot(h, We2_ref[...], preferred_element_type=jnp.float32) + be2_ref[...]
    z = ts_ref[...] * Wt_ref[...] + bt_ref[...]
    ef = ef * (1.0 / (1.0 + jnp.exp(-z)))
    pre = (gx_ref[...]
           + jnp.dot(ef, Wn1b_ref[...], preferred_element_type=jnp.float32)
           + bn1_ref[...])
    msg_ref[...] = (jnp.dot(jnp.maximum(pre, 0.0), Wn2_ref[...],
                            preferred_element_type=jnp.float32) + bn2_ref[...])


def _combine_body(acc_ref, x_ref, o_ref):
    o_ref[...] = acc_ref[0] + acc_ref[1] + x_ref[...]


def _make_gather(N, D, E):
    per_tile = E // (N_TILES * CH)
    mesh = plsc.VectorSubcoreMesh(core_axis_name="c", subcore_axis_name="s")

    @functools.partial(
        pl.kernel,
        out_type=jax.ShapeDtypeStruct((E, D), jnp.float32),
        mesh=mesh,
        scratch_types=[
            pltpu.VMEM((per_tile, CH), jnp.int32),
            pltpu.VMEM((CH, D), jnp.float32),
            pltpu.SemaphoreType.DMA,
        ],
    )
    def gather_k(xa_hbm, rows_hbm, gx_hbm, idx_v, buf_v, sem):
        c = lax.axis_index("c")
        s = lax.axis_index("s")
        w = c * NUM_TEC + s
        chunk0 = w * per_tile
        pltpu.sync_copy(rows_hbm.at[w], idx_v)

        def body(j, carry):
            pltpu.async_copy(xa_hbm.at[idx_v.at[j]], buf_v, sem).wait()
            pltpu.sync_copy(buf_v, gx_hbm.at[pl.ds((chunk0 + j) * CH, CH)])
            return carry

        lax.fori_loop(0, per_tile, body, 0)

    return gather_k


def _make_scatter(N, D, E, NP):
    per_tile = E // (N_TILES * CH)
    rows_nt = NP // NUM_TEC
    wb = 128
    mesh = plsc.VectorSubcoreMesh(core_axis_name="c", subcore_axis_name="s")

    @functools.partial(
        pl.kernel,
        out_type=jax.ShapeDtypeStruct((NUM_SC, NP, D), jnp.float32),
        mesh=mesh,
        scratch_types=[
            pltpu.VMEM((per_tile, CH), jnp.int32),
            pltpu.VMEM((CH, D), jnp.float32),
            pltpu.VMEM((wb, D), jnp.float32),
            pltpu.VMEM_SHARED((NP, D), jnp.float32),
            pltpu.SemaphoreType.DMA,
        ],
    )
    def scatter_k(msg_hbm, cols_hbm, z_hbm, acc_hbm, col_v, buf_v, wbuf_v,
                  acc_sh, sem):
        c = lax.axis_index("c")
        s = lax.axis_index("s")

        pltpu.sync_copy(z_hbm, wbuf_v)

        def zbody(k, carry):
            pltpu.sync_copy(wbuf_v, acc_sh.at[pl.ds(s * rows_nt + k * wb, wb)])
            return carry

        lax.fori_loop(0, rows_nt // wb, zbody, 0)
        plsc.subcore_barrier()

        w = c * NUM_TEC + s
        chunk0 = w * per_tile
        pltpu.sync_copy(cols_hbm.at[w], col_v)

        def body(j, carry):
            pltpu.sync_copy(msg_hbm.at[pl.ds((chunk0 + j) * CH, CH)], buf_v)
            pltpu.sync_copy(buf_v, acc_sh.at[col_v.at[j]], add=True)
            return carry

        lax.fori_loop(0, per_tile, body, 0)
        plsc.subcore_barrier()

        def wbody(k, carry):
            r0 = s * rows_nt + k * wb
            pltpu.sync_copy(acc_sh.at[pl.ds(r0, wb)], wbuf_v)
            pltpu.sync_copy(wbuf_v, acc_hbm.at[c, pl.ds(r0, wb)])
            return carry

        lax.fori_loop(0, rows_nt // wb, wbody, 0)

    return scatter_k


def kernel(x, edge_index, edge_attr, timestamps,
           We1, be1, We2, be2, Wn1, bn1, Wn2, bn2, Wt, bt):
    N, D = x.shape
    E, DE = edge_attr.shape
    per_tile = E // (N_TILES * CH)
    NP = 10240
    row = edge_index[0].reshape(N_TILES, per_tile, CH)
    col = edge_index[1].reshape(N_TILES, per_tile, CH)
    ts2 = timestamps.reshape(E, 1)
    Wn1a = Wn1[:D]
    Wn1b = Wn1[D:]
    H = We1.shape[1]

    BN = 1000
    xa = pl.pallas_call(
        _xa_body,
        grid=(N // BN,),
        in_specs=[
            pl.BlockSpec((BN, D), lambda i: (i, 0)),
            pl.BlockSpec((D, D), lambda i: (0, 0)),
        ],
        out_specs=pl.BlockSpec((BN, D), lambda i: (i, 0)),
        out_shape=jax.ShapeDtypeStruct((N, D), jnp.float32),
    )(x, Wn1a)

    gx = _make_gather(N, D, E)(xa, row)

    BE = 1280
    full = lambda a: pl.BlockSpec(a.shape, lambda i: tuple(0 for _ in a.shape))
    msg = pl.pallas_call(
        _msg_body,
        grid=(E // BE,),
        in_specs=[
            pl.BlockSpec((BE, D), lambda i: (i, 0)),
            pl.BlockSpec((BE, DE), lambda i: (i, 0)),
            pl.BlockSpec((BE, 1), lambda i: (i, 0)),
            full(We1), full(be1.reshape(1, H)),
            full(We2), full(be2.reshape(1, D)),
            full(Wt), full(bt.reshape(1, D)),
            full(Wn1b), full(bn1.reshape(1, D)),
            full(Wn2), full(bn2.reshape(1, D)),
        ],
        out_specs=pl.BlockSpec((BE, D), lambda i: (i, 0)),
        out_shape=jax.ShapeDtypeStruct((E, D), jnp.float32),
    )(gx, edge_attr, ts2, We1, be1.reshape(1, H), We2, be2.reshape(1, D),
      Wt, bt.reshape(1, D), Wn1b, bn1.reshape(1, D), Wn2, bn2.reshape(1, D))

    zeros = jnp.zeros((128, D), jnp.float32)
    acc = _make_scatter(N, D, E, NP)(msg, col, zeros)

    out = pl.pallas_call(
        _combine_body,
        grid=(N // BN,),
        in_specs=[
            pl.BlockSpec((NUM_SC, BN, D), lambda i: (0, i, 0)),
            pl.BlockSpec((BN, D), lambda i: (i, 0)),
        ],
        out_specs=pl.BlockSpec((BN, D), lambda i: (i, 0)),
        out_shape=jax.ShapeDtypeStruct((N, D), jnp.float32),
    )(acc, x)
    return out


# speedup vs baseline: 2.4445x; 1.0347x over previous
import functools

import jax
import jax.numpy as jnp
from jax import lax
from jax.experimental import pallas as pl
from jax.experimental.pallas import tpu as pltpu
from jax.experimental.pallas import tpu_sc as plsc

NUM_SC = 2
NUM_TEC = 16
N_TILES = NUM_SC * NUM_TEC
CH = 80
NB = 5


def _xa_body(x_ref, w_ref, o_ref):
    o_ref[...] = jnp.dot(x_ref[...], w_ref[...], preferred_element_type=jnp.float32)


def _msg_body(gx_ref, ea_ref, ts_ref, We1_ref, be1_ref, We2_ref, be2_ref,
              Wt_ref, bt_ref, Wn1b_ref, bn1_ref, Wn2_ref, bn2_ref, msg_ref):
    h = jnp.maximum(
        jnp.dot(ea_ref[...], We1_ref[...], preferred_element_type=jnp.float32)
        + be1_ref[...], 0.0)
    ef = jnp.dot(h, We2_ref[...], preferred_element_type=jnp.float32) + be2_ref[...]
    z = ts_ref[...] * Wt_ref[...] + bt_ref[...]
    ef = ef * (1.0 / (1.0 + jnp.exp(-z)))
    pre = (gx_ref[...]
           + jnp.dot(ef, Wn1b_ref[...], preferred_element_type=jnp.float32)
           + bn1_ref[...])
    msg_ref[...] = (jnp.dot(jnp.maximum(pre, 0.0), Wn2_ref[...],
                            preferred_element_type=jnp.float32) + bn2_ref[...])


def _combine_body(acc_ref, x_ref, o_ref):
    o_ref[...] = acc_ref[0] + acc_ref[1] + x_ref[...]


def _make_gather(N, D, E):
    per_tile = E // (N_TILES * CH)
    mesh = plsc.VectorSubcoreMesh(core_axis_name="c", subcore_axis_name="s")

    @functools.partial(
        pl.kernel,
        out_type=jax.ShapeDtypeStruct((E, D), jnp.float32),
        mesh=mesh,
        scratch_types=[
            pltpu.VMEM((per_tile, CH), jnp.int32),
            pltpu.VMEM((NB, CH, D), jnp.float32),
            pltpu.SemaphoreType.DMA((NB,)),
            pltpu.SemaphoreType.DMA((NB,)),
        ],
    )
    def gather_k(xa_hbm, rows_hbm, gx_hbm, idx_v, buf_v, gsem, wsem):
        c = lax.axis_index("c")
        s = lax.axis_index("s")
        w = c * NUM_TEC + s
        chunk0 = w * per_tile
        pltpu.sync_copy(rows_hbm.at[w], idx_v)

        for b in range(NB):
            pltpu.async_copy(xa_hbm.at[idx_v.at[b]], buf_v.at[b], gsem.at[b])

        def group(g, carry):
            for b in range(NB):
                ci = g * NB + b
                pltpu.make_async_copy(
                    xa_hbm.at[idx_v.at[ci]], buf_v.at[b], gsem.at[b]).wait()
                dst = gx_hbm.at[pl.ds((chunk0 + ci) * CH, CH)]
                pltpu.async_copy(buf_v.at[b], dst, wsem.at[b])
                pltpu.make_async_copy(buf_v.at[b], dst, wsem.at[b]).wait()
                nxt = ci + NB

                @pl.when(nxt < per_tile)
                def _():
                    pltpu.async_copy(
                        xa_hbm.at[idx_v.at[nxt]], buf_v.at[b], gsem.at[b])

            return carry

        lax.fori_loop(0, per_tile // NB, group, 0)

    return gather_k


def _make_scatter(N, D, E, NP):
    per_tile = E // (N_TILES * CH)
    rows_nt = NP // NUM_TEC
    wb = 128
    nbs = 2
    mesh = plsc.VectorSubcoreMesh(core_axis_name="c", subcore_axis_name="s")

    @functools.partial(
        pl.kernel,
        out_type=jax.ShapeDtypeStruct((NUM_SC, NP, D), jnp.float32),
        mesh=mesh,
        scratch_types=[
            pltpu.VMEM((per_tile, CH), jnp.int32),
            pltpu.VMEM((nbs, CH, D), jnp.float32),
            pltpu.VMEM_SHARED((NP, D), jnp.float32),
            pltpu.SemaphoreType.DMA((nbs,)),
        ],
    )
    def scatter_k(msg_hbm, cols_hbm, z_hbm, acc_hbm, col_v, buf_v,
                  acc_sh, lsem):
        c = lax.axis_index("c")
        s = lax.axis_index("s")

        def zbody(k, carry):
            pltpu.sync_copy(z_hbm, acc_sh.at[pl.ds(s * rows_nt + k * wb, wb)])
            return carry

        lax.fori_loop(0, rows_nt // wb, zbody, 0)

        w = c * NUM_TEC + s
        chunk0 = w * per_tile
        pltpu.sync_copy(cols_hbm.at[w], col_v)
        plsc.subcore_barrier()

        for b in range(nbs):
            src = msg_hbm.at[pl.ds((chunk0 + b) * CH, CH)]
            pltpu.async_copy(src, buf_v.at[b], lsem.at[b])

        def group(g, carry):
            for b in range(nbs):
                ci = g * nbs + b
                src = msg_hbm.at[pl.ds((chunk0 + ci) * CH, CH)]
                pltpu.make_async_copy(src, buf_v.at[b], lsem.at[b]).wait()
                pltpu.sync_copy(buf_v.at[b], acc_sh.at[col_v.at[ci]], add=True)
                nxt = ci + nbs

                @pl.when(nxt < per_tile)
                def _():
                    pltpu.async_copy(
                        msg_hbm.at[pl.ds((chunk0 + nxt) * CH, CH)],
                        buf_v.at[b], lsem.at[b])

            return carry

        lax.fori_loop(0, per_tile // nbs, group, 0)
        for ci in range((per_tile // nbs) * nbs, per_tile):
            b = ci % nbs
            src = msg_hbm.at[pl.ds((chunk0 + ci) * CH, CH)]
            pltpu.make_async_copy(src, buf_v.at[b], lsem.at[b]).wait()
            pltpu.sync_copy(buf_v.at[b], acc_sh.at[col_v.at[ci]], add=True)
        plsc.subcore_barrier()

        def wbody(k, carry):
            r0 = s * rows_nt + k * wb
            pltpu.sync_copy(acc_sh.at[pl.ds(r0, wb)], acc_hbm.at[c, pl.ds(r0, wb)])
            return carry

        lax.fori_loop(0, rows_nt // wb, wbody, 0)

    return scatter_k


def kernel(x, edge_index, edge_attr, timestamps,
           We1, be1, We2, be2, Wn1, bn1, Wn2, bn2, Wt, bt):
    N, D = x.shape
    E, DE = edge_attr.shape
    per_tile = E // (N_TILES * CH)
    NP = 10240
    row = edge_index[0].reshape(N_TILES, per_tile, CH)
    col = edge_index[1].reshape(N_TILES, per_tile, CH)
    ts2 = timestamps.reshape(E, 1)
    Wn1a = Wn1[:D]
    Wn1b = Wn1[D:]
    H = We1.shape[1]

    BN = 1000
    xa = pl.pallas_call(
        _xa_body,
        grid=(N // BN,),
        in_specs=[
            pl.BlockSpec((BN, D), lambda i: (i, 0)),
            pl.BlockSpec((D, D), lambda i: (0, 0)),
        ],
        out_specs=pl.BlockSpec((BN, D), lambda i: (i, 0)),
        out_shape=jax.ShapeDtypeStruct((N, D), jnp.float32),
    )(x, Wn1a)

    gx = _make_gather(N, D, E)(xa, row)

    BE = 1280
    full = lambda a: pl.BlockSpec(a.shape, lambda i: tuple(0 for _ in a.shape))
    msg = pl.pallas_call(
        _msg_body,
        grid=(E // BE,),
        in_specs=[
            pl.BlockSpec((BE, D), lambda i: (i, 0)),
            pl.BlockSpec((BE, DE), lambda i: (i, 0)),
            pl.BlockSpec((BE, 1), lambda i: (i, 0)),
            full(We1), full(be1.reshape(1, H)),
            full(We2), full(be2.reshape(1, D)),
            full(Wt), full(bt.reshape(1, D)),
            full(Wn1b), full(bn1.reshape(1, D)),
            full(Wn2), full(bn2.reshape(1, D)),
        ],
        out_specs=pl.BlockSpec((BE, D), lambda i: (i, 0)),
        out_shape=jax.ShapeDtypeStruct((E, D), jnp.float32),
    )(gx, edge_attr, ts2, We1, be1.reshape(1, H), We2, be2.reshape(1, D),
      Wt, bt.reshape(1, D), Wn1b, bn1.reshape(1, D), Wn2, bn2.reshape(1, D))

    zeros = jnp.zeros((128, D), jnp.float32)
    acc = _make_scatter(N, D, E, NP)(msg, col, zeros)

    out = pl.pallas_call(
        _combine_body,
        grid=(N // BN,),
        in_specs=[
            pl.BlockSpec((NUM_SC, BN, D), lambda i: (0, i, 0)),
            pl.BlockSpec((BN, D), lambda i: (i, 0)),
        ],
        out_specs=pl.BlockSpec((BN, D), lambda i: (i, 0)),
        out_shape=jax.ShapeDtypeStruct((N, D), jnp.float32),
    )(acc, x)
    return out
